# baseline (speedup 1.0000x reference)
import functools

import jax
import jax.numpy as jnp
from jax import lax
from jax.experimental import pallas as pl
from jax.experimental.pallas import tpu as pltpu
from jax.experimental.pallas import tpu_sc as plsc

H = 1024
W = 1024
NB = 8
K = NB ** 3
SIGMA_C = 16.0
NPIX = H * W

NC = 2
NS = 16
NW = NC * NS
CHUNK = NPIX // NW
VECS = CHUNK // 16

TC_GRID = 8
TC_CHUNK = NPIX // TC_GRID


def _minmax_kernel(img_ref, edges_ref, acc_min, acc_max):
    i = pl.program_id(0)
    x = img_ref[...]
    m = jnp.min(x, axis=0, keepdims=True)
    M = jnp.max(x, axis=0, keepdims=True)

    @pl.when(i == 0)
    def _():
        acc_min[...] = m
        acc_max[...] = M

    @pl.when(i > 0)
    def _():
        acc_min[...] = jnp.minimum(acc_min[...], m)
        acc_max[...] = jnp.maximum(acc_max[...], M)

    @pl.when(i == TC_GRID - 1)
    def _():
        col = lax.broadcasted_iota(jnp.int32, (1, 3 * W), 1)
        chan = col - (col // 3) * 3
        ii = lax.broadcasted_iota(jnp.int32, (1, NB), 1).astype(jnp.float32)
        for c in range(3):
            sel = chan == c
            lo = jnp.min(jnp.where(sel, acc_min[...], jnp.inf))
            hi = jnp.max(jnp.where(sel, acc_max[...], -jnp.inf))
            step = (hi - lo) / NB
            edges_ref[c:c + 1, :] = lo + step * ii


def _compute_edges(image_rows):
    rows = H // TC_GRID
    return pl.pallas_call(
        _minmax_kernel,
        grid=(TC_GRID,),
        in_specs=[pl.BlockSpec((rows, 3 * W), lambda i: (i, 0))],
        out_specs=pl.BlockSpec((3, NB), lambda i: (0, 0)),
        out_shape=jax.ShapeDtypeStruct((3, NB), jnp.float32),
        scratch_shapes=[
            pltpu.VMEM((1, 3 * W), jnp.float32),
            pltpu.VMEM((1, 3 * W), jnp.float32),
        ],
    )(image_rows)


def _binidx_kernel(img_ref, edges_ref, out_ref):
    v = img_ref[...]
    e = edges_ref[...]
    acc = jnp.zeros(v.shape, jnp.int32)
    for i in range(1, NB):
        acc = acc + (v >= e[:, i:i + 1]).astype(jnp.int32)
    flat = acc[0:1] * (NB * NB) + acc[1:2] * NB + acc[2:3]
    out_ref[...] = flat.reshape(TC_CHUNK)


def _compute_flat_idx(img_t, edges):
    return pl.pallas_call(
        _binidx_kernel,
        grid=(TC_GRID,),
        in_specs=[
            pl.BlockSpec((3, TC_CHUNK), lambda i: (0, i)),
            pl.BlockSpec((3, NB), lambda i: (0, 0)),
        ],
        out_specs=pl.BlockSpec((TC_CHUNK,), lambda i: (i,)),
        out_shape=jax.ShapeDtypeStruct((NPIX,), jnp.int32),
    )(img_t, edges)


def _sc_hist_body(idx_hbm, out_hbm, idxbuf, t0, t1, t2, t3, t4, red):
    wid = lax.axis_index("s") * NC + lax.axis_index("c")
    base = wid * CHUNK
    pltpu.sync_copy(idx_hbm.at[pl.ds(base, CHUNK)], idxbuf)

    tabs = (t0, t1, t2, t3, t4)
    zf = jnp.zeros((16,), jnp.float32)

    def zero_body(j, carry):
        off = j * 16
        for t in tabs:
            t[pl.ds(off, 16)] = zf
        return carry

    lax.fori_loop(0, 16 * K // 16, zero_body, 0)

    lane = lax.iota(jnp.int32, 16)
    laneoff = lane * K
    ones = jnp.ones((16,), jnp.float32)

    UNROLL = 8

    def body(i, carry):
        for u in range(UNROLL):
            ii = i * UNROLL + u
            idxv = idxbuf[pl.ds(ii * 16, 16)]
            vaddr = idxv + laneoff
            p = base + ii * 16 + lane
            xf = jnp.bitwise_and(p, W - 1).astype(jnp.float32)
            yf = lax.shift_right_logical(p, 10).astype(jnp.float32)
            plsc.addupdate_scatter(t0, [vaddr], ones)
            plsc.addupdate_scatter(t1, [vaddr], xf)
            plsc.addupdate_scatter(t2, [vaddr], yf)
            plsc.addupdate_scatter(t3, [vaddr], xf * xf)
            plsc.addupdate_scatter(t4, [vaddr], yf * yf)
        return carry

    lax.fori_loop(0, VECS // UNROLL, body, 0)

    for q, t in enumerate(tabs):
        def red_body(j, carry, t=t):
            off = j * 16
            acc = t[pl.ds(off, 16)]
            for l in range(1, 16):
                acc = acc + t[pl.ds(l * K + off, 16)]
            red[pl.ds(off, 16)] = acc
            return carry

        lax.fori_loop(0, K // 16, red_body, 0)
        pltpu.sync_copy(red, out_hbm.at[q, wid])


def _sc_hist(flat_idx):
    mesh = plsc.VectorSubcoreMesh(core_axis_name="c", subcore_axis_name="s", num_cores=NC, num_subcores=NS)
    fn = functools.partial(
        pl.kernel,
        out_type=jax.ShapeDtypeStruct((5, NW, K), jnp.float32),
        mesh=mesh,
        compiler_params=pltpu.CompilerParams(needs_layout_passes=False),
        scratch_types=[
            pltpu.VMEM((CHUNK,), jnp.int32),
            pltpu.VMEM((16 * K,), jnp.float32),
            pltpu.VMEM((16 * K,), jnp.float32),
            pltpu.VMEM((16 * K,), jnp.float32),
            pltpu.VMEM((16 * K,), jnp.float32),
            pltpu.VMEM((16 * K,), jnp.float32),
            pltpu.VMEM((K,), jnp.float32),
        ],
    )(_sc_hist_body)
    return fn(flat_idx)


def _dense_kernel(part_ref, part_t_ref, edges_ref, mean_ref, cov_ref, out_ref):
    part = part_ref[...]
    hist = jnp.sum(part[0], axis=0, keepdims=True)
    sx = jnp.sum(part[1], axis=0, keepdims=True)
    sy = jnp.sum(part[2], axis=0, keepdims=True)
    sx2 = jnp.sum(part[3], axis=0, keepdims=True)
    sy2 = jnp.sum(part[4], axis=0, keepdims=True)
    hist_col = jnp.sum(part_t_ref[:, 0, :], axis=1, keepdims=True)

    kr = lax.broadcasted_iota(jnp.int32, (1, K), 1)
    kc = lax.broadcasted_iota(jnp.int32, (K, 1), 0)
    d2 = jnp.zeros((K, K), jnp.float32)
    for c, shift in ((0, 6), (1, 3), (2, 0)):
        lo = edges_ref[c, 0]
        step = (edges_ref[c, 1] - edges_ref[c, 0])
        qr = jnp.bitwise_and(lax.shift_right_logical(kr, shift), NB - 1)
        qc = jnp.bitwise_and(lax.shift_right_logical(kc, shift), NB - 1)
        ur = lo + step * qr.astype(jnp.float32)
        uc = lo + step * qc.astype(jnp.float32)
        diff = uc - ur
        d2 = d2 + diff * diff

    mask_row = (hist > 0.0).astype(jnp.float32)
    mask_col = (hist_col > 0.0).astype(jnp.float32)
    pair = mask_col * mask_row
    E = jnp.exp(-d2 / (2.0 * SIGMA_C ** 2)) * pair
    D = jnp.sqrt(jnp.maximum(d2, 0.0)) * pair

    def bf(x):
        return x.astype(jnp.bfloat16).astype(jnp.float32)

    def matvec_col(M, row):
        return jnp.sum(bf(M) * bf(row), axis=1, keepdims=True)

    contrast = matvec_col(D, hist)
    norm_arr = jnp.maximum(matvec_col(E, hist), 1e-8)
    mx = matvec_col(E, sx) / norm_arr
    my = matvec_col(E, sy) / norm_arr
    mx2 = matvec_col(E, sx2) / norm_arr
    my2 = matvec_col(E, sy2) / norm_arr
    vx = jnp.maximum(mx2 - mx * mx, 0.0)
    vy = jnp.maximum(my2 - my * my, 0.0)

    g0 = jnp.sqrt(12.0 * vx) / float(W)
    g1 = jnp.sqrt(12.0 * vy) / float(H)
    g2 = (mx - W / 2.0) / float(W)
    g3 = (my - H / 2.0) / float(H)
    X = (g0 - mean_ref[0], g1 - mean_ref[1], g2 - mean_ref[2], g3 - mean_ref[3])
    maha = jnp.zeros((K, 1), jnp.float32)
    for j in range(4):
        t = bf(X[0]) * bf(cov_ref[0, j])
        for i in range(1, 4):
            t = t + bf(X[i]) * bf(cov_ref[i, j])
        maha = maha + t * X[j]
    shape_prob = jnp.exp(-maha / 2.0)

    sal_col = contrast * shape_prob * mask_col
    smooth_num = jnp.sum(bf(E) * bf(sal_col), axis=0, keepdims=True)
    norm2 = jnp.maximum(jnp.sum(E, axis=0, keepdims=True), 1e-8)
    smoothed = smooth_num / norm2

    maskb = hist > 0.0
    min_sal = jnp.min(jnp.where(maskb, smoothed, jnp.inf))
    max_sal = jnp.max(jnp.where(maskb, smoothed, -jnp.inf))
    scale = 255.0 / jnp.maximum(max_sal - min_sal, 1e-8)
    normalized = jnp.where(max_sal > min_sal,
                           scale * (smoothed - min_sal),
                           jnp.zeros_like(smoothed))
    out_ref[...] = normalized * mask_row


def _dense_stage(partials, edges, mean_vector, cov_inv):
    part_t = jnp.transpose(partials, (2, 0, 1))
    return pl.pallas_call(
        _dense_kernel,
        in_specs=[
            pl.BlockSpec(memory_space=pltpu.MemorySpace.VMEM),
            pl.BlockSpec(memory_space=pltpu.MemorySpace.VMEM),
            pl.BlockSpec(memory_space=pltpu.MemorySpace.SMEM),
            pl.BlockSpec(memory_space=pltpu.MemorySpace.SMEM),
            pl.BlockSpec(memory_space=pltpu.MemorySpace.SMEM),
        ],
        out_shape=jax.ShapeDtypeStruct((1, K), jnp.float32),
    )(partials, part_t, edges, mean_vector, cov_inv)


def _sc_gather_body(idx_hbm, norm_hbm, out_hbm, idxbuf, tab, obuf):
    wid = lax.axis_index("s") * NC + lax.axis_index("c")
    base = wid * CHUNK
    pltpu.sync_copy(idx_hbm.at[pl.ds(base, CHUNK)], idxbuf)
    pltpu.sync_copy(norm_hbm, tab)

    UNROLL = 8

    def body(i, carry):
        for u in range(UNROLL):
            ii = i * UNROLL + u
            idxv = idxbuf[pl.ds(ii * 16, 16)]
            obuf[pl.ds(ii * 16, 16)] = plsc.load_gather(tab, [idxv])
        return carry

    lax.fori_loop(0, VECS // UNROLL, body, 0)
    pltpu.sync_copy(obuf, out_hbm.at[pl.ds(base, CHUNK)])


def _sc_gather(flat_idx, normalized):
    mesh = plsc.VectorSubcoreMesh(core_axis_name="c", subcore_axis_name="s", num_cores=NC, num_subcores=NS)
    fn = functools.partial(
        pl.kernel,
        out_type=jax.ShapeDtypeStruct((NPIX,), jnp.float32),
        mesh=mesh,
        compiler_params=pltpu.CompilerParams(needs_layout_passes=False),
        scratch_types=[
            pltpu.VMEM((CHUNK,), jnp.int32),
            pltpu.VMEM((K,), jnp.float32),
            pltpu.VMEM((CHUNK,), jnp.float32),
        ],
    )(_sc_gather_body)
    return fn(flat_idx, normalized)


def kernel(image, mean_vector, cov_inv):
    img_t = jnp.transpose(image.reshape(NPIX, 3))
    edges = _compute_edges(image.reshape(H, 3 * W))
    flat_idx = _compute_flat_idx(img_t, edges)
    partials = _sc_hist(flat_idx)
    normalized = _dense_stage(partials, edges, mean_vector, cov_inv)
    sal_map = _sc_gather(flat_idx, normalized.reshape(K))
    return sal_map.reshape(H, W)


# speedup vs baseline: 1.4974x; 1.4974x over previous
import functools

import jax
import jax.numpy as jnp
from jax import lax
from jax.experimental import pallas as pl
from jax.experimental.pallas import tpu as pltpu
from jax.experimental.pallas import tpu_sc as plsc

H = 1024
W = 1024
NB = 8
K = NB ** 3
SIGMA_C = 16.0
NPIX = H * W

NC = 2
NS = 16
NW = NC * NS
CHUNK = NPIX // NW
VECS = CHUNK // 16

TC_GRID = 8
TC_CHUNK = NPIX // TC_GRID


def _minmax_kernel(img_ref, edges_ref, acc_min, acc_max):
    i = pl.program_id(0)
    x = img_ref[...]
    m = jnp.min(x, axis=1, keepdims=True)
    M = jnp.max(x, axis=1, keepdims=True)

    @pl.when(i == 0)
    def _():
        acc_min[...] = jnp.broadcast_to(m, acc_min.shape)
        acc_max[...] = jnp.broadcast_to(M, acc_max.shape)

    @pl.when(i > 0)
    def _():
        acc_min[...] = jnp.minimum(acc_min[...], m)
        acc_max[...] = jnp.maximum(acc_max[...], M)

    @pl.when(i == TC_GRID - 1)
    def _():
        lo = jnp.min(acc_min[...], axis=1, keepdims=True)
        hi = jnp.max(acc_max[...], axis=1, keepdims=True)
        step = (hi - lo) / NB
        ii = lax.broadcasted_iota(jnp.int32, (3, NB), 1).astype(jnp.float32)
        edges_ref[...] = lo + step * ii


def _compute_edges(img_t):
    return pl.pallas_call(
        _minmax_kernel,
        grid=(TC_GRID,),
        in_specs=[pl.BlockSpec((3, TC_CHUNK), lambda i: (0, i))],
        out_specs=pl.BlockSpec((3, NB), lambda i: (0, 0)),
        out_shape=jax.ShapeDtypeStruct((3, NB), jnp.float32),
        scratch_shapes=[
            pltpu.VMEM((3, 128), jnp.float32),
            pltpu.VMEM((3, 128), jnp.float32),
        ],
    )(img_t)


def _binidx_kernel(img_ref, edges_ref, out_ref):
    v = img_ref[...]
    e = edges_ref[...]
    acc = jnp.zeros(v.shape, jnp.int32)
    for i in range(1, NB):
        acc = acc + (v >= e[:, i:i + 1]).astype(jnp.int32)
    flat = acc[0:1] * (NB * NB) + acc[1:2] * NB + acc[2:3]
    out_ref[...] = flat.reshape(TC_CHUNK)


def _compute_flat_idx(img_t, edges):
    return pl.pallas_call(
        _binidx_kernel,
        grid=(TC_GRID,),
        in_specs=[
            pl.BlockSpec((3, TC_CHUNK), lambda i: (0, i)),
            pl.BlockSpec((3, NB), lambda i: (0, 0)),
        ],
        out_specs=pl.BlockSpec((TC_CHUNK,), lambda i: (i,)),
        out_shape=jax.ShapeDtypeStruct((NPIX,), jnp.int32),
    )(img_t, edges)


def _sc_hist_body(idx_hbm, out_hbm, idxbuf, t0, t1, t2, t3, t4, red):
    wid = lax.axis_index("s") * NC + lax.axis_index("c")
    base = wid * CHUNK
    pltpu.sync_copy(idx_hbm.at[pl.ds(base, CHUNK)], idxbuf)

    tabs = (t0, t1, t2, t3, t4)
    zf = jnp.zeros((16,), jnp.float32)

    @plsc.parallel_loop(0, 16 * K // 16, unroll=4)
    def _(j):
        off = j * 16
        for t in tabs:
            t[pl.ds(off, 16)] = zf

    lane = lax.iota(jnp.int32, 16)
    laneoff = lane * K
    ones = jnp.ones((16,), jnp.float32)

    @plsc.parallel_loop(0, VECS, unroll=4)
    def _(i):
        idxv = idxbuf[pl.ds(i * 16, 16)]
        vaddr = idxv + laneoff
        p = base + i * 16 + lane
        xf = jnp.bitwise_and(p, W - 1).astype(jnp.float32)
        yf = lax.shift_right_logical(p, 10).astype(jnp.float32)
        plsc.addupdate_scatter(t0, [vaddr], ones)
        plsc.addupdate_scatter(t1, [vaddr], xf)
        plsc.addupdate_scatter(t2, [vaddr], yf)
        plsc.addupdate_scatter(t3, [vaddr], xf * xf)
        plsc.addupdate_scatter(t4, [vaddr], yf * yf)

    for q, t in enumerate(tabs):
        @plsc.parallel_loop(0, K // 16, unroll=2)
        def _(j, t=t):
            off = j * 16
            acc = t[pl.ds(off, 16)]
            for l in range(1, 16):
                acc = acc + t[pl.ds(l * K + off, 16)]
            red[pl.ds(off, 16)] = acc

        pltpu.sync_copy(red, out_hbm.at[q, wid])


def _sc_hist(flat_idx):
    mesh = plsc.VectorSubcoreMesh(core_axis_name="c", subcore_axis_name="s", num_cores=NC, num_subcores=NS)
    fn = functools.partial(
        pl.kernel,
        out_type=jax.ShapeDtypeStruct((5, NW, K), jnp.float32),
        mesh=mesh,
        compiler_params=pltpu.CompilerParams(needs_layout_passes=False),
        scratch_types=[
            pltpu.VMEM((CHUNK,), jnp.int32),
            pltpu.VMEM((16 * K,), jnp.float32),
            pltpu.VMEM((16 * K,), jnp.float32),
            pltpu.VMEM((16 * K,), jnp.float32),
            pltpu.VMEM((16 * K,), jnp.float32),
            pltpu.VMEM((16 * K,), jnp.float32),
            pltpu.VMEM((K,), jnp.float32),
        ],
    )(_sc_hist_body)
    return fn(flat_idx)


def _dense_kernel(part_ref, part_t_ref, edges_ref, mean_ref, cov_ref, out_ref):
    part = part_ref[...]
    hist = jnp.sum(part[0], axis=0, keepdims=True)
    sx = jnp.sum(part[1], axis=0, keepdims=True)
    sy = jnp.sum(part[2], axis=0, keepdims=True)
    sx2 = jnp.sum(part[3], axis=0, keepdims=True)
    sy2 = jnp.sum(part[4], axis=0, keepdims=True)
    hist_col = jnp.sum(part_t_ref[:, 0, :], axis=1, keepdims=True)

    kr = lax.broadcasted_iota(jnp.int32, (1, K), 1)
    kc = lax.broadcasted_iota(jnp.int32, (K, 1), 0)
    d2 = jnp.zeros((K, K), jnp.float32)
    for c, shift in ((0, 6), (1, 3), (2, 0)):
        lo = edges_ref[c, 0]
        step = (edges_ref[c, 1] - edges_ref[c, 0])
        qr = jnp.bitwise_and(lax.shift_right_logical(kr, shift), NB - 1)
        qc = jnp.bitwise_and(lax.shift_right_logical(kc, shift), NB - 1)
        ur = lo + step * qr.astype(jnp.float32)
        uc = lo + step * qc.astype(jnp.float32)
        diff = uc - ur
        d2 = d2 + diff * diff

    mask_row = (hist > 0.0).astype(jnp.float32)
    mask_col = (hist_col > 0.0).astype(jnp.float32)
    pair = mask_col * mask_row
    E = jnp.exp(-d2 / (2.0 * SIGMA_C ** 2)) * pair
    D = jnp.sqrt(jnp.maximum(d2, 0.0)) * pair

    def bf(x):
        return x.astype(jnp.bfloat16).astype(jnp.float32)

    def matvec_col(M, row):
        return jnp.sum(bf(M) * bf(row), axis=1, keepdims=True)

    contrast = matvec_col(D, hist)
    norm_arr = jnp.maximum(matvec_col(E, hist), 1e-8)
    mx = matvec_col(E, sx) / norm_arr
    my = matvec_col(E, sy) / norm_arr
    mx2 = matvec_col(E, sx2) / norm_arr
    my2 = matvec_col(E, sy2) / norm_arr
    vx = jnp.maximum(mx2 - mx * mx, 0.0)
    vy = jnp.maximum(my2 - my * my, 0.0)

    g0 = jnp.sqrt(12.0 * vx) / float(W)
    g1 = jnp.sqrt(12.0 * vy) / float(H)
    g2 = (mx - W / 2.0) / float(W)
    g3 = (my - H / 2.0) / float(H)
    X = (g0 - mean_ref[0], g1 - mean_ref[1], g2 - mean_ref[2], g3 - mean_ref[3])
    maha = jnp.zeros((K, 1), jnp.float32)
    for j in range(4):
        t = bf(X[0]) * bf(cov_ref[0, j])
        for i in range(1, 4):
            t = t + bf(X[i]) * bf(cov_ref[i, j])
        maha = maha + t * X[j]
    shape_prob = jnp.exp(-maha / 2.0)

    sal_col = contrast * shape_prob * mask_col
    smooth_num = jnp.sum(bf(E) * bf(sal_col), axis=0, keepdims=True)
    norm2 = jnp.maximum(jnp.sum(E, axis=0, keepdims=True), 1e-8)
    smoothed = smooth_num / norm2

    maskb = hist > 0.0
    min_sal = jnp.min(jnp.where(maskb, smoothed, jnp.inf))
    max_sal = jnp.max(jnp.where(maskb, smoothed, -jnp.inf))
    scale = 255.0 / jnp.maximum(max_sal - min_sal, 1e-8)
    normalized = jnp.where(max_sal > min_sal,
                           scale * (smoothed - min_sal),
                           jnp.zeros_like(smoothed))
    out_ref[...] = normalized * mask_row


def _dense_stage(partials, edges, mean_vector, cov_inv):
    part_t = jnp.transpose(partials, (2, 0, 1))
    return pl.pallas_call(
        _dense_kernel,
        in_specs=[
            pl.BlockSpec(memory_space=pltpu.MemorySpace.VMEM),
            pl.BlockSpec(memory_space=pltpu.MemorySpace.VMEM),
            pl.BlockSpec(memory_space=pltpu.MemorySpace.SMEM),
            pl.BlockSpec(memory_space=pltpu.MemorySpace.SMEM),
            pl.BlockSpec(memory_space=pltpu.MemorySpace.SMEM),
        ],
        out_shape=jax.ShapeDtypeStruct((1, K), jnp.float32),
    )(partials, part_t, edges, mean_vector, cov_inv)


def _sc_gather_body(idx_hbm, norm_hbm, out_hbm, idxbuf, tab, obuf):
    wid = lax.axis_index("s") * NC + lax.axis_index("c")
    base = wid * CHUNK
    pltpu.sync_copy(idx_hbm.at[pl.ds(base, CHUNK)], idxbuf)
    pltpu.sync_copy(norm_hbm, tab)

    @plsc.parallel_loop(0, VECS, unroll=8)
    def _(i):
        idxv = idxbuf[pl.ds(i * 16, 16)]
        obuf[pl.ds(i * 16, 16)] = plsc.load_gather(tab, [idxv])
    pltpu.sync_copy(obuf, out_hbm.at[pl.ds(base, CHUNK)])


def _sc_gather(flat_idx, normalized):
    mesh = plsc.VectorSubcoreMesh(core_axis_name="c", subcore_axis_name="s", num_cores=NC, num_subcores=NS)
    fn = functools.partial(
        pl.kernel,
        out_type=jax.ShapeDtypeStruct((NPIX,), jnp.float32),
        mesh=mesh,
        compiler_params=pltpu.CompilerParams(needs_layout_passes=False),
        scratch_types=[
            pltpu.VMEM((CHUNK,), jnp.int32),
            pltpu.VMEM((K,), jnp.float32),
            pltpu.VMEM((CHUNK,), jnp.float32),
        ],
    )(_sc_gather_body)
    return fn(flat_idx, normalized)


def kernel(image, mean_vector, cov_inv):
    img_t = jnp.transpose(image.reshape(NPIX, 3))
    edges = _compute_edges(img_t)
    flat_idx = _compute_flat_idx(img_t, edges)
    partials = _sc_hist(flat_idx)
    normalized = _dense_stage(partials, edges, mean_vector, cov_inv)
    sal_map = _sc_gather(flat_idx, normalized.reshape(K))
    return sal_map.reshape(H, W)
